# baseline (speedup 1.0000x reference)
import jax
import jax.numpy as jnp
from jax import lax
from jax.experimental import pallas as pl
from jax.experimental.pallas import tpu as pltpu
from jax.experimental.pallas import tpu_sc as plsc

N = 10000
CDIM = 128
GDIM = 64

EO = 330000
NTILES = 32
NS = 16
SB = 64
NSB = 5184
EP = NSB * SB
RT0 = 178
RT1 = 146
DROWS = 80
ZCH = 125

BN = 2000
GRID = N // BN

_HIGH = jax.lax.Precision.HIGHEST


def _embed_body(x_ref, w_ref, atts_ref, attd_ref, h_ref, as_ref, ad_ref):
    h = jnp.dot(x_ref[...], w_ref[...], preferred_element_type=jnp.float32,
                precision=_HIGH)
    h_ref[...] = h
    as_ref[...] = jnp.sum(h * atts_ref[...], axis=1, keepdims=True)
    ad_ref[...] = jnp.sum(h * attd_ref[...], axis=1, keepdims=True)


def _embed(x, W, atts, attd):
    return pl.pallas_call(
        _embed_body,
        grid=(GRID,),
        in_specs=[
            pl.BlockSpec((BN, CDIM), lambda i: (i, 0)),
            pl.BlockSpec((CDIM, CDIM), lambda i: (0, 0)),
            pl.BlockSpec((1, CDIM), lambda i: (0, 0)),
            pl.BlockSpec((1, CDIM), lambda i: (0, 0)),
        ],
        out_specs=[
            pl.BlockSpec((BN, CDIM), lambda i: (i, 0)),
            pl.BlockSpec((BN, 1), lambda i: (i, 0)),
            pl.BlockSpec((BN, 1), lambda i: (i, 0)),
        ],
        out_shape=[
            jax.ShapeDtypeStruct((N, CDIM), jnp.float32),
            jax.ShapeDtypeStruct((N, 1), jnp.float32),
            jax.ShapeDtypeStruct((N, 1), jnp.float32),
        ],
    )(x, W, atts, attd)


def _combine(p, d):
    m = p[0] + p[1]
    den = d[0] + d[1]
    return m / (den + 1e-16)


def _combine_embed_body(p_ref, d_ref, b_ref, w_ref, atts_ref, attd_ref,
                        h_ref, as_ref, ad_ref):
    xc = jnp.maximum(_combine(p_ref[...], d_ref[...]) + b_ref[...], 0.0)
    h = jnp.dot(xc, w_ref[...], preferred_element_type=jnp.float32,
                precision=_HIGH)
    h_ref[...] = h
    as_ref[...] = jnp.sum(h * atts_ref[...], axis=1, keepdims=True)
    ad_ref[...] = jnp.sum(h * attd_ref[...], axis=1, keepdims=True)


def _combine_embed(p, d, b, W, atts, attd):
    return pl.pallas_call(
        _combine_embed_body,
        grid=(GRID,),
        in_specs=[
            pl.BlockSpec((2, BN, CDIM), lambda i: (0, i, 0)),
            pl.BlockSpec((2, BN, 1), lambda i: (0, i, 0)),
            pl.BlockSpec((1, CDIM), lambda i: (0, 0)),
            pl.BlockSpec((CDIM, CDIM), lambda i: (0, 0)),
            pl.BlockSpec((1, CDIM), lambda i: (0, 0)),
            pl.BlockSpec((1, CDIM), lambda i: (0, 0)),
        ],
        out_specs=[
            pl.BlockSpec((BN, CDIM), lambda i: (i, 0)),
            pl.BlockSpec((BN, 1), lambda i: (i, 0)),
            pl.BlockSpec((BN, 1), lambda i: (i, 0)),
        ],
        out_shape=[
            jax.ShapeDtypeStruct((N, CDIM), jnp.float32),
            jax.ShapeDtypeStruct((N, 1), jnp.float32),
            jax.ShapeDtypeStruct((N, 1), jnp.float32),
        ],
    )(p, d, b, W, atts, attd)


def _pool_body(p_ref, d_ref, b_ref, batch_ref, linw_ref, linb_ref, out_ref,
               pool_acc, cnt_acc):
    i = pl.program_id(0)

    @pl.when(i == 0)
    def _():
        pool_acc[...] = jnp.zeros((GDIM, CDIM), jnp.float32)
        cnt_acc[...] = jnp.zeros((GDIM, CDIM), jnp.float32)

    xc = jnp.maximum(_combine(p_ref[...], d_ref[...]) + b_ref[...], 0.0)
    bt = batch_ref[...]
    gid = lax.broadcasted_iota(jnp.int32, (BN, GDIM), 1)
    oneh = (bt == gid).astype(jnp.float32)
    psum = lax.dot_general(oneh, xc, (((0,), (0,)), ((), ())),
                           preferred_element_type=jnp.float32,
                           precision=_HIGH)
    ones = jnp.ones((BN, CDIM), jnp.float32)
    csum = lax.dot_general(oneh, ones, (((0,), (0,)), ((), ())),
                           preferred_element_type=jnp.float32,
                           precision=_HIGH)
    pool_acc[...] += psum
    cnt_acc[...] += csum

    @pl.when(i == GRID - 1)
    def _():
        pooled = pool_acc[...] / jnp.maximum(cnt_acc[...], 1.0)
        out_ref[...] = jnp.dot(pooled, linw_ref[...],
                               preferred_element_type=jnp.float32,
                               precision=_HIGH) + linb_ref[...]


def _pool(p, d, b, batch, linW, linb):
    return pl.pallas_call(
        _pool_body,
        grid=(GRID,),
        in_specs=[
            pl.BlockSpec((2, BN, CDIM), lambda i: (0, i, 0)),
            pl.BlockSpec((2, BN, 1), lambda i: (0, i, 0)),
            pl.BlockSpec((1, CDIM), lambda i: (0, 0)),
            pl.BlockSpec((BN, 1), lambda i: (i, 0)),
            pl.BlockSpec((CDIM, CDIM), lambda i: (0, 0)),
            pl.BlockSpec((1, CDIM), lambda i: (0, 0)),
        ],
        out_specs=pl.BlockSpec((GDIM, CDIM), lambda i: (0, 0)),
        out_shape=jax.ShapeDtypeStruct((GDIM, CDIM), jnp.float32),
        scratch_shapes=[
            pltpu.VMEM((GDIM, CDIM), jnp.float32),
            pltpu.VMEM((GDIM, CDIM), jnp.float32),
        ],
    )(p, d, b, batch, linW, linb)


def _agg_body(src_hbm, dst_hbm, h_hbm, as_hbm, ad_hbm,
              outm_hbm, outd_hbm,
              srow0, srow1, drow0, drow1, asv, adv, rin0, rin1, wb0, wb1,
              denv, iden, sx0, sx1, gsem0, gsem1, isem0, isem1,
              ssem0, ssem1, accm, accd):
    c = lax.axis_index("c")
    s = lax.axis_index("s")
    start = jnp.where(c == 0, s * RT0, NS * RT0 + s * RT1)
    nb = jnp.where(c == 0, RT0, RT1)
    lane = lax.iota(jnp.int32, 16)
    lane0 = lane == 0

    pltpu.sync_copy(as_hbm, asv)
    pltpu.sync_copy(ad_hbm, adv)

    pltpu.sync_copy(src_hbm.at[start, 0], srow0)
    pltpu.sync_copy(dst_hbm.at[start, 0], drow0)
    pltpu.async_copy(src_hbm.at[start + 1, 0], srow1, isem1)
    pltpu.async_copy(dst_hbm.at[start + 1, 0], drow1, isem1)
    pltpu.async_copy(h_hbm.at[srow0], rin0, gsem0)

    def zden(r, _):
        for g in range(8):
            denv[r, pl.ds(g * 16, 16)] = jnp.zeros((16,), jnp.float32)
        return 0
    lax.fori_loop(0, DROWS, zden, 0)
    for i in range(DROWS // 16):
        iden[pl.ds(i * 16, 16)] = i * 16 + lane

    nz = (ZCH - s + NS - 1) // NS

    def zacc(i, _):
        t = s + i * NS
        pltpu.sync_copy(denv, accm.at[pl.ds(t * DROWS, DROWS)])
        return 0
    lax.fori_loop(0, nz, zacc, 0)

    @pl.when(s == 0)
    def _():
        pltpu.sync_copy(denv, accd)

    plsc.subcore_barrier()

    def compute_w(b, srw, wbk):
        for g in range(SB // 16):
            sidx = srw[pl.ds(g * 16, 16)]
            didx = (drow0 if srw is srow0 else drow1)[pl.ds(g * 16, 16)]
            e = plsc.load_gather(asv, [sidx]) + plsc.load_gather(adv, [didx])
            e = jnp.where(e > 0, e, e * jnp.float32(0.2))
            w = jnp.exp(e)
            pos = (start + b) * SB + g * 16 + lane
            wbk[pl.ds(g * 16, 16)] = jnp.where(pos < EO, w, jnp.float32(0.0))

    def scale_rows(rin_k, wbk):
        @plsc.parallel_loop(0, SB, unroll=8)
        def _(r):
            ridx = jnp.full((16,), r, jnp.int32)
            wspl = plsc.load_gather(wbk, [ridx])
            for g in range(8):
                rin_k[r, pl.ds(g * 16, 16)] = (
                    rin_k[r, pl.ds(g * 16, 16)] * wspl)

    def denom_rows(wbk, drw):
        @plsc.parallel_loop(0, SB, unroll=8)
        def _(r):
            ridx = jnp.full((16,), r, jnp.int32)
            wspl = plsc.load_gather(wbk, [ridx])
            dsp = plsc.load_gather(drw, [ridx])
            plsc.addupdate_scatter(
                denv,
                [lax.shift_right_logical(dsp, 7),
                 lax.bitwise_and(dsp, jnp.int32(127))],
                wspl, mask=lane0)

    bufs = ((srow0, drow0, rin0, wb0, sx0, gsem0, isem0, ssem0),
            (srow1, drow1, rin1, wb1, sx1, gsem1, isem1, ssem1))

    def body(jj, _):
        for k in range(2):
            b = jj * 2 + k
            srw, drw, rin_k, wbk, sxk, gsem_k, isem_k, ssem_k = bufs[k]
            srw_o, drw_o, rin_o, _, sxo, gsem_o, isem_o, ssem_o = bufs[1 - k]

            compute_w(b, srw, wbk)

            @pl.when(b + 1 < nb)
            def _():
                pltpu.make_async_copy(src_hbm.at[start + b + 1, 0], srw_o,
                                      isem_o).wait()
                pltpu.make_async_copy(dst_hbm.at[start + b + 1, 0], drw_o,
                                      isem_o).wait()

                @pl.when(b >= 1)
                def _():
                    pltpu.make_async_copy(rin_o, accm.at[sxo], ssem_o).wait()

                pltpu.async_copy(h_hbm.at[srw_o], rin_o, gsem_o)

            pltpu.make_async_copy(h_hbm.at[srw], rin_k, gsem_k).wait()
            scale_rows(rin_k, wbk)
            for g in range(SB // 16):
                sxk[pl.ds(g * 16, 16)] = drw[pl.ds(g * 16, 16)]
            pltpu.async_copy(rin_k, accm.at[sxk], ssem_k, add=True)
            denom_rows(wbk, drw)

            @pl.when(b + 2 < nb)
            def _():
                pltpu.async_copy(src_hbm.at[start + b + 2, 0], srw, isem_k)
                pltpu.async_copy(dst_hbm.at[start + b + 2, 0], drw, isem_k)
        return 0

    lax.fori_loop(0, nb // 2, body, 0)
    pltpu.make_async_copy(rin0, accm.at[sx0], ssem0).wait()
    pltpu.make_async_copy(rin1, accm.at[sx1], ssem1).wait()

    pltpu.sync_copy(denv, accd.at[iden], add=True)

    plsc.subcore_barrier()

    def cout(i, _):
        t = s + i * NS
        pltpu.sync_copy(accm.at[pl.ds(t * DROWS, DROWS)],
                        outm_hbm.at[c, pl.ds(t * DROWS, DROWS)])
        return 0
    lax.fori_loop(0, nz, cout, 0)

    @pl.when(s == 0)
    def _():
        pltpu.sync_copy(accd, outd_hbm.at[c])


_agg_call = pl.kernel(
    _agg_body,
    out_type=(jax.ShapeDtypeStruct((2, N, CDIM), jnp.float32),
              jax.ShapeDtypeStruct((2, DROWS, 128), jnp.float32)),
    mesh=plsc.VectorSubcoreMesh(core_axis_name="c", subcore_axis_name="s"),
    compiler_params=pltpu.CompilerParams(needs_layout_passes=False),
    scratch_types=[
        pltpu.VMEM((SB,), jnp.int32),
        pltpu.VMEM((SB,), jnp.int32),
        pltpu.VMEM((SB,), jnp.int32),
        pltpu.VMEM((SB,), jnp.int32),
        pltpu.VMEM((N,), jnp.float32),
        pltpu.VMEM((N,), jnp.float32),
        pltpu.VMEM((SB, CDIM), jnp.float32),
        pltpu.VMEM((SB, CDIM), jnp.float32),
        pltpu.VMEM((SB,), jnp.float32),
        pltpu.VMEM((SB,), jnp.float32),
        pltpu.VMEM((DROWS, 128), jnp.float32),
        pltpu.VMEM((DROWS,), jnp.int32),
        pltpu.VMEM((SB,), jnp.int32),
        pltpu.VMEM((SB,), jnp.int32),
        pltpu.SemaphoreType.DMA,
        pltpu.SemaphoreType.DMA,
        pltpu.SemaphoreType.DMA,
        pltpu.SemaphoreType.DMA,
        pltpu.SemaphoreType.DMA,
        pltpu.SemaphoreType.DMA,
        pltpu.VMEM_SHARED((N, CDIM), jnp.float32),
        pltpu.VMEM_SHARED((DROWS, 128), jnp.float32),
    ],
)


@jax.jit
def kernel(x, edge_index, batch, W1, att_src1, att_dst1, b1,
           W2, att_src2, att_dst2, b2, lin_W, lin_b):
    loop = jnp.arange(N, dtype=edge_index.dtype)
    src = jnp.concatenate([edge_index[0], loop])
    dst = jnp.concatenate([edge_index[1], loop])
    srcp = jnp.pad(src, (0, EP - EO)).reshape(NSB, 1, SB)
    dstp = jnp.pad(dst, (0, EP - EO)).reshape(NSB, 1, SB)

    h1, a_s1, a_d1 = _embed(x, W1, att_src1.reshape(1, CDIM),
                            att_dst1.reshape(1, CDIM))
    p1, d1 = _agg_call(srcp, dstp, h1, a_s1.reshape(N), a_d1.reshape(N))
    h2, a_s2, a_d2 = _combine_embed(p1, d1.reshape(2, DROWS * 128, 1),
                                    b1.reshape(1, CDIM), W2,
                                    att_src2.reshape(1, CDIM),
                                    att_dst2.reshape(1, CDIM))
    p2, d2 = _agg_call(srcp, dstp, h2, a_s2.reshape(N), a_d2.reshape(N))
    return _pool(p2, d2.reshape(2, DROWS * 128, 1), b2.reshape(1, CDIM),
                 batch.reshape(N, 1), lin_W, lin_b.reshape(1, CDIM))


# speedup vs baseline: 1.0182x; 1.0182x over previous
import jax
import jax.numpy as jnp
from jax import lax
from jax.experimental import pallas as pl
from jax.experimental.pallas import tpu as pltpu
from jax.experimental.pallas import tpu_sc as plsc

N = 10000
CDIM = 128
GDIM = 64

EO = 330000
NTILES = 32
NS = 16
SB = 64
NSB = 5184
EP = NSB * SB
RT0 = 184
RT1 = 140
DROWS = 80
ZCH = 125

BN = 2000
GRID = N // BN

_HIGH = jax.lax.Precision.HIGHEST


def _embed_body(x_ref, w_ref, atts_ref, attd_ref, h_ref, as_ref, ad_ref):
    h = jnp.dot(x_ref[...], w_ref[...], preferred_element_type=jnp.float32,
                precision=_HIGH)
    h_ref[...] = h
    as_ref[...] = jnp.sum(h * atts_ref[...], axis=1, keepdims=True)
    ad_ref[...] = jnp.sum(h * attd_ref[...], axis=1, keepdims=True)


def _embed(x, W, atts, attd):
    return pl.pallas_call(
        _embed_body,
        grid=(GRID,),
        in_specs=[
            pl.BlockSpec((BN, CDIM), lambda i: (i, 0)),
            pl.BlockSpec((CDIM, CDIM), lambda i: (0, 0)),
            pl.BlockSpec((1, CDIM), lambda i: (0, 0)),
            pl.BlockSpec((1, CDIM), lambda i: (0, 0)),
        ],
        out_specs=[
            pl.BlockSpec((BN, CDIM), lambda i: (i, 0)),
            pl.BlockSpec((BN, 1), lambda i: (i, 0)),
            pl.BlockSpec((BN, 1), lambda i: (i, 0)),
        ],
        out_shape=[
            jax.ShapeDtypeStruct((N, CDIM), jnp.float32),
            jax.ShapeDtypeStruct((N, 1), jnp.float32),
            jax.ShapeDtypeStruct((N, 1), jnp.float32),
        ],
    )(x, W, atts, attd)


def _combine(p, d):
    m = p[0] + p[1]
    den = d[0] + d[1]
    return m / (den + 1e-16)


def _combine_embed_body(p_ref, d_ref, b_ref, w_ref, atts_ref, attd_ref,
                        h_ref, as_ref, ad_ref):
    xc = jnp.maximum(_combine(p_ref[...], d_ref[...]) + b_ref[...], 0.0)
    h = jnp.dot(xc, w_ref[...], preferred_element_type=jnp.float32,
                precision=_HIGH)
    h_ref[...] = h
    as_ref[...] = jnp.sum(h * atts_ref[...], axis=1, keepdims=True)
    ad_ref[...] = jnp.sum(h * attd_ref[...], axis=1, keepdims=True)


def _combine_embed(p, d, b, W, atts, attd):
    return pl.pallas_call(
        _combine_embed_body,
        grid=(GRID,),
        in_specs=[
            pl.BlockSpec((2, BN, CDIM), lambda i: (0, i, 0)),
            pl.BlockSpec((2, BN, 1), lambda i: (0, i, 0)),
            pl.BlockSpec((1, CDIM), lambda i: (0, 0)),
            pl.BlockSpec((CDIM, CDIM), lambda i: (0, 0)),
            pl.BlockSpec((1, CDIM), lambda i: (0, 0)),
            pl.BlockSpec((1, CDIM), lambda i: (0, 0)),
        ],
        out_specs=[
            pl.BlockSpec((BN, CDIM), lambda i: (i, 0)),
            pl.BlockSpec((BN, 1), lambda i: (i, 0)),
            pl.BlockSpec((BN, 1), lambda i: (i, 0)),
        ],
        out_shape=[
            jax.ShapeDtypeStruct((N, CDIM), jnp.float32),
            jax.ShapeDtypeStruct((N, 1), jnp.float32),
            jax.ShapeDtypeStruct((N, 1), jnp.float32),
        ],
    )(p, d, b, W, atts, attd)


def _pool_body(p_ref, d_ref, b_ref, batch_ref, linw_ref, linb_ref, out_ref,
               pool_acc, cnt_acc):
    i = pl.program_id(0)

    @pl.when(i == 0)
    def _():
        pool_acc[...] = jnp.zeros((GDIM, CDIM), jnp.float32)
        cnt_acc[...] = jnp.zeros((GDIM, CDIM), jnp.float32)

    xc = jnp.maximum(_combine(p_ref[...], d_ref[...]) + b_ref[...], 0.0)
    bt = batch_ref[...]
    gid = lax.broadcasted_iota(jnp.int32, (BN, GDIM), 1)
    oneh = (bt == gid).astype(jnp.float32)
    psum = lax.dot_general(oneh, xc, (((0,), (0,)), ((), ())),
                           preferred_element_type=jnp.float32,
                           precision=_HIGH)
    ones = jnp.ones((BN, CDIM), jnp.float32)
    csum = lax.dot_general(oneh, ones, (((0,), (0,)), ((), ())),
                           preferred_element_type=jnp.float32,
                           precision=_HIGH)
    pool_acc[...] += psum
    cnt_acc[...] += csum

    @pl.when(i == GRID - 1)
    def _():
        pooled = pool_acc[...] / jnp.maximum(cnt_acc[...], 1.0)
        out_ref[...] = jnp.dot(pooled, linw_ref[...],
                               preferred_element_type=jnp.float32,
                               precision=_HIGH) + linb_ref[...]


def _pool(p, d, b, batch, linW, linb):
    return pl.pallas_call(
        _pool_body,
        grid=(GRID,),
        in_specs=[
            pl.BlockSpec((2, BN, CDIM), lambda i: (0, i, 0)),
            pl.BlockSpec((2, BN, 1), lambda i: (0, i, 0)),
            pl.BlockSpec((1, CDIM), lambda i: (0, 0)),
            pl.BlockSpec((BN, 1), lambda i: (i, 0)),
            pl.BlockSpec((CDIM, CDIM), lambda i: (0, 0)),
            pl.BlockSpec((1, CDIM), lambda i: (0, 0)),
        ],
        out_specs=pl.BlockSpec((GDIM, CDIM), lambda i: (0, 0)),
        out_shape=jax.ShapeDtypeStruct((GDIM, CDIM), jnp.float32),
        scratch_shapes=[
            pltpu.VMEM((GDIM, CDIM), jnp.float32),
            pltpu.VMEM((GDIM, CDIM), jnp.float32),
        ],
    )(p, d, b, batch, linW, linb)


def _agg_body(src_hbm, dst_hbm, h_hbm, as_hbm, ad_hbm,
              outm_hbm, outd_hbm,
              srow0, srow1, drow0, drow1, asv, adv, rin0, rin1, wb0, wb1,
              denv, iden, sx0, sx1, gsem0, gsem1, isem0, isem1,
              ssem0, ssem1, accm, accd):
    c = lax.axis_index("c")
    s = lax.axis_index("s")
    start = jnp.where(c == 0, s * RT0, NS * RT0 + s * RT1)
    nb = jnp.where(c == 0, RT0, RT1)
    lane = lax.iota(jnp.int32, 16)
    lane0 = lane == 0

    pltpu.sync_copy(as_hbm, asv)
    pltpu.sync_copy(ad_hbm, adv)

    pltpu.sync_copy(src_hbm.at[start, 0], srow0)
    pltpu.sync_copy(dst_hbm.at[start, 0], drow0)
    pltpu.async_copy(src_hbm.at[start + 1, 0], srow1, isem1)
    pltpu.async_copy(dst_hbm.at[start + 1, 0], drow1, isem1)
    pltpu.async_copy(h_hbm.at[srow0], rin0, gsem0)

    def zden(r, _):
        for g in range(8):
            denv[r, pl.ds(g * 16, 16)] = jnp.zeros((16,), jnp.float32)
        return 0
    lax.fori_loop(0, DROWS, zden, 0)
    for i in range(DROWS // 16):
        iden[pl.ds(i * 16, 16)] = i * 16 + lane

    nz = (ZCH - s + NS - 1) // NS

    def zacc(i, _):
        t = s + i * NS
        pltpu.sync_copy(denv, accm.at[pl.ds(t * DROWS, DROWS)])
        return 0
    lax.fori_loop(0, nz, zacc, 0)

    @pl.when(s == 0)
    def _():
        pltpu.sync_copy(denv, accd)

    plsc.subcore_barrier()

    def compute_w(b, srw, wbk):
        for g in range(SB // 16):
            sidx = srw[pl.ds(g * 16, 16)]
            didx = (drow0 if srw is srow0 else drow1)[pl.ds(g * 16, 16)]
            e = plsc.load_gather(asv, [sidx]) + plsc.load_gather(adv, [didx])
            e = jnp.where(e > 0, e, e * jnp.float32(0.2))
            w = jnp.exp(e)
            pos = (start + b) * SB + g * 16 + lane
            wbk[pl.ds(g * 16, 16)] = jnp.where(pos < EO, w, jnp.float32(0.0))

    def scale_rows(rin_k, wbk):
        @plsc.parallel_loop(0, SB, unroll=16)
        def _(r):
            ridx = jnp.full((16,), r, jnp.int32)
            wspl = plsc.load_gather(wbk, [ridx])
            for g in range(8):
                rin_k[r, pl.ds(g * 16, 16)] = (
                    rin_k[r, pl.ds(g * 16, 16)] * wspl)

    def denom_rows(wbk, drw):
        @plsc.parallel_loop(0, SB, unroll=4)
        def _(r):
            ridx = jnp.full((16,), r, jnp.int32)
            wspl = plsc.load_gather(wbk, [ridx])
            dsp = plsc.load_gather(drw, [ridx])
            plsc.addupdate_scatter(
                denv,
                [lax.shift_right_logical(dsp, 7),
                 lax.bitwise_and(dsp, jnp.int32(127))],
                wspl, mask=lane0)

    bufs = ((srow0, drow0, rin0, wb0, sx0, gsem0, isem0, ssem0),
            (srow1, drow1, rin1, wb1, sx1, gsem1, isem1, ssem1))

    def body(jj, _):
        for k in range(2):
            b = jj * 2 + k
            srw, drw, rin_k, wbk, sxk, gsem_k, isem_k, ssem_k = bufs[k]
            srw_o, drw_o, rin_o, _, sxo, gsem_o, isem_o, ssem_o = bufs[1 - k]

            compute_w(b, srw, wbk)

            @pl.when(b + 1 < nb)
            def _():
                pltpu.make_async_copy(src_hbm.at[start + b + 1, 0], srw_o,
                                      isem_o).wait()
                pltpu.make_async_copy(dst_hbm.at[start + b + 1, 0], drw_o,
                                      isem_o).wait()

                @pl.when(b >= 1)
                def _():
                    pltpu.make_async_copy(rin_o, accm.at[sxo], ssem_o).wait()

                pltpu.async_copy(h_hbm.at[srw_o], rin_o, gsem_o)

            pltpu.make_async_copy(h_hbm.at[srw], rin_k, gsem_k).wait()
            scale_rows(rin_k, wbk)
            for g in range(SB // 16):
                sxk[pl.ds(g * 16, 16)] = drw[pl.ds(g * 16, 16)]
            pltpu.async_copy(rin_k, accm.at[sxk], ssem_k, add=True)
            denom_rows(wbk, drw)

            @pl.when(b + 2 < nb)
            def _():
                pltpu.async_copy(src_hbm.at[start + b + 2, 0], srw, isem_k)
                pltpu.async_copy(dst_hbm.at[start + b + 2, 0], drw, isem_k)
        return 0

    lax.fori_loop(0, nb // 2, body, 0)
    pltpu.make_async_copy(rin0, accm.at[sx0], ssem0).wait()
    pltpu.make_async_copy(rin1, accm.at[sx1], ssem1).wait()

    pltpu.sync_copy(denv, accd.at[iden], add=True)

    plsc.subcore_barrier()

    def cout(i, _):
        t = s + i * NS
        pltpu.sync_copy(accm.at[pl.ds(t * DROWS, DROWS)],
                        outm_hbm.at[c, pl.ds(t * DROWS, DROWS)])
        return 0
    lax.fori_loop(0, nz, cout, 0)

    @pl.when(s == 0)
    def _():
        pltpu.sync_copy(accd, outd_hbm.at[c])


_agg_call = pl.kernel(
    _agg_body,
    out_type=(jax.ShapeDtypeStruct((2, N, CDIM), jnp.float32),
              jax.ShapeDtypeStruct((2, DROWS, 128), jnp.float32)),
    mesh=plsc.VectorSubcoreMesh(core_axis_name="c", subcore_axis_name="s"),
    compiler_params=pltpu.CompilerParams(needs_layout_passes=False),
    scratch_types=[
        pltpu.VMEM((SB,), jnp.int32),
        pltpu.VMEM((SB,), jnp.int32),
        pltpu.VMEM((SB,), jnp.int32),
        pltpu.VMEM((SB,), jnp.int32),
        pltpu.VMEM((N,), jnp.float32),
        pltpu.VMEM((N,), jnp.float32),
        pltpu.VMEM((SB, CDIM), jnp.float32),
        pltpu.VMEM((SB, CDIM), jnp.float32),
        pltpu.VMEM((SB,), jnp.float32),
        pltpu.VMEM((SB,), jnp.float32),
        pltpu.VMEM((DROWS, 128), jnp.float32),
        pltpu.VMEM((DROWS,), jnp.int32),
        pltpu.VMEM((SB,), jnp.int32),
        pltpu.VMEM((SB,), jnp.int32),
        pltpu.SemaphoreType.DMA,
        pltpu.SemaphoreType.DMA,
        pltpu.SemaphoreType.DMA,
        pltpu.SemaphoreType.DMA,
        pltpu.SemaphoreType.DMA,
        pltpu.SemaphoreType.DMA,
        pltpu.VMEM_SHARED((N, CDIM), jnp.float32),
        pltpu.VMEM_SHARED((DROWS, 128), jnp.float32),
    ],
)


@jax.jit
def kernel(x, edge_index, batch, W1, att_src1, att_dst1, b1,
           W2, att_src2, att_dst2, b2, lin_W, lin_b):
    loop = jnp.arange(N, dtype=edge_index.dtype)
    src = jnp.concatenate([edge_index[0], loop])
    dst = jnp.concatenate([edge_index[1], loop])
    srcp = jnp.pad(src, (0, EP - EO)).reshape(NSB, 1, SB)
    dstp = jnp.pad(dst, (0, EP - EO)).reshape(NSB, 1, SB)

    h1, a_s1, a_d1 = _embed(x, W1, att_src1.reshape(1, CDIM),
                            att_dst1.reshape(1, CDIM))
    p1, d1 = _agg_call(srcp, dstp, h1, a_s1.reshape(N), a_d1.reshape(N))
    h2, a_s2, a_d2 = _combine_embed(p1, d1.reshape(2, DROWS * 128, 1),
                                    b1.reshape(1, CDIM), W2,
                                    att_src2.reshape(1, CDIM),
                                    att_dst2.reshape(1, CDIM))
    p2, d2 = _agg_call(srcp, dstp, h2, a_s2.reshape(N), a_d2.reshape(N))
    return _pool(p2, d2.reshape(2, DROWS * 128, 1), b2.reshape(1, CDIM),
                 batch.reshape(N, 1), lin_W, lin_b.reshape(1, CDIM))


# speedup vs baseline: 1.0788x; 1.0594x over previous
import jax
import jax.numpy as jnp
from jax import lax
from jax.experimental import pallas as pl
from jax.experimental.pallas import tpu as pltpu
from jax.experimental.pallas import tpu_sc as plsc

N = 10000
CDIM = 128
GDIM = 64

EO = 330000
NTILES = 32
NS = 16
SB = 64
NSB = 5184
EP = NSB * SB
RT0 = 186
RT1 = 138
DROWS = 80
ZCH = 125

BN = 2000
GRID = N // BN

_HIGH = jax.lax.Precision.HIGHEST


def _embed_body(x_ref, w_ref, atts_ref, attd_ref, h_ref, as_ref, ad_ref):
    h = jnp.dot(x_ref[...], w_ref[...], preferred_element_type=jnp.float32,
                precision=_HIGH)
    h_ref[...] = h
    as_ref[...] = jnp.sum(h * atts_ref[...], axis=1, keepdims=True)
    ad_ref[...] = jnp.sum(h * attd_ref[...], axis=1, keepdims=True)


def _embed(x, W, atts, attd):
    return pl.pallas_call(
        _embed_body,
        grid=(GRID,),
        in_specs=[
            pl.BlockSpec((BN, CDIM), lambda i: (i, 0)),
            pl.BlockSpec((CDIM, CDIM), lambda i: (0, 0)),
            pl.BlockSpec((1, CDIM), lambda i: (0, 0)),
            pl.BlockSpec((1, CDIM), lambda i: (0, 0)),
        ],
        out_specs=[
            pl.BlockSpec((BN, CDIM), lambda i: (i, 0)),
            pl.BlockSpec((BN, 1), lambda i: (i, 0)),
            pl.BlockSpec((BN, 1), lambda i: (i, 0)),
        ],
        out_shape=[
            jax.ShapeDtypeStruct((N, CDIM), jnp.float32),
            jax.ShapeDtypeStruct((N, 1), jnp.float32),
            jax.ShapeDtypeStruct((N, 1), jnp.float32),
        ],
    )(x, W, atts, attd)


def _combine(p, d):
    m = p[0] + p[1]
    den = d[0] + d[1]
    return m / (den + 1e-16)


def _combine_embed_body(p_ref, d_ref, b_ref, w_ref, atts_ref, attd_ref,
                        h_ref, as_ref, ad_ref):
    xc = jnp.maximum(_combine(p_ref[...], d_ref[...]) + b_ref[...], 0.0)
    h = jnp.dot(xc, w_ref[...], preferred_element_type=jnp.float32,
                precision=_HIGH)
    h_ref[...] = h
    as_ref[...] = jnp.sum(h * atts_ref[...], axis=1, keepdims=True)
    ad_ref[...] = jnp.sum(h * attd_ref[...], axis=1, keepdims=True)


def _combine_embed(p, d, b, W, atts, attd):
    return pl.pallas_call(
        _combine_embed_body,
        grid=(GRID,),
        in_specs=[
            pl.BlockSpec((2, BN, CDIM), lambda i: (0, i, 0)),
            pl.BlockSpec((2, BN, 1), lambda i: (0, i, 0)),
            pl.BlockSpec((1, CDIM), lambda i: (0, 0)),
            pl.BlockSpec((CDIM, CDIM), lambda i: (0, 0)),
            pl.BlockSpec((1, CDIM), lambda i: (0, 0)),
            pl.BlockSpec((1, CDIM), lambda i: (0, 0)),
        ],
        out_specs=[
            pl.BlockSpec((BN, CDIM), lambda i: (i, 0)),
            pl.BlockSpec((BN, 1), lambda i: (i, 0)),
            pl.BlockSpec((BN, 1), lambda i: (i, 0)),
        ],
        out_shape=[
            jax.ShapeDtypeStruct((N, CDIM), jnp.float32),
            jax.ShapeDtypeStruct((N, 1), jnp.float32),
            jax.ShapeDtypeStruct((N, 1), jnp.float32),
        ],
    )(p, d, b, W, atts, attd)


def _pool_body(p_ref, d_ref, b_ref, batch_ref, linw_ref, linb_ref, out_ref,
               pool_acc, cnt_acc):
    i = pl.program_id(0)

    @pl.when(i == 0)
    def _():
        pool_acc[...] = jnp.zeros((GDIM, CDIM), jnp.float32)
        cnt_acc[...] = jnp.zeros((GDIM, CDIM), jnp.float32)

    xc = jnp.maximum(_combine(p_ref[...], d_ref[...]) + b_ref[...], 0.0)
    bt = batch_ref[...]
    gid = lax.broadcasted_iota(jnp.int32, (BN, GDIM), 1)
    oneh = (bt == gid).astype(jnp.float32)
    psum = lax.dot_general(oneh, xc, (((0,), (0,)), ((), ())),
                           preferred_element_type=jnp.float32,
                           precision=_HIGH)
    ones = jnp.ones((BN, CDIM), jnp.float32)
    csum = lax.dot_general(oneh, ones, (((0,), (0,)), ((), ())),
                           preferred_element_type=jnp.float32,
                           precision=_HIGH)
    pool_acc[...] += psum
    cnt_acc[...] += csum

    @pl.when(i == GRID - 1)
    def _():
        pooled = pool_acc[...] / jnp.maximum(cnt_acc[...], 1.0)
        out_ref[...] = jnp.dot(pooled, linw_ref[...],
                               preferred_element_type=jnp.float32,
                               precision=_HIGH) + linb_ref[...]


def _pool(p, d, b, batch, linW, linb):
    return pl.pallas_call(
        _pool_body,
        grid=(GRID,),
        in_specs=[
            pl.BlockSpec((2, BN, CDIM), lambda i: (0, i, 0)),
            pl.BlockSpec((2, BN, 1), lambda i: (0, i, 0)),
            pl.BlockSpec((1, CDIM), lambda i: (0, 0)),
            pl.BlockSpec((BN, 1), lambda i: (i, 0)),
            pl.BlockSpec((CDIM, CDIM), lambda i: (0, 0)),
            pl.BlockSpec((1, CDIM), lambda i: (0, 0)),
        ],
        out_specs=pl.BlockSpec((GDIM, CDIM), lambda i: (0, 0)),
        out_shape=jax.ShapeDtypeStruct((GDIM, CDIM), jnp.float32),
        scratch_shapes=[
            pltpu.VMEM((GDIM, CDIM), jnp.float32),
            pltpu.VMEM((GDIM, CDIM), jnp.float32),
        ],
    )(p, d, b, batch, linW, linb)


def _agg_body(src_hbm, dst_hbm, h_hbm, ap_hbm,
              outm_hbm, outd_hbm,
              srow0, srow1, srow2, drow0, drow1, drow2, apv,
              rin0, rin1, rin2, wb0, wb1, wb2, denv, iden, sx0, sx1, sx2,
              gsem0, gsem1, gsem2, isem0, isem1, isem2,
              ssem0, ssem1, ssem2, accm, accd):
    c = lax.axis_index("c")
    s = lax.axis_index("s")
    start = jnp.where(c == 0, s * RT0, NS * RT0 + s * RT1)
    nb = jnp.where(c == 0, RT0, RT1)
    lane = lax.iota(jnp.int32, 16)
    lane0 = lane == 0

    pltpu.sync_copy(ap_hbm, apv)

    pltpu.sync_copy(src_hbm.at[start, 0], srow0)
    pltpu.sync_copy(dst_hbm.at[start, 0], drow0)
    pltpu.sync_copy(src_hbm.at[start + 1, 0], srow1)
    pltpu.sync_copy(dst_hbm.at[start + 1, 0], drow1)
    pltpu.async_copy(src_hbm.at[start + 2, 0], srow2, isem2)
    pltpu.async_copy(dst_hbm.at[start + 2, 0], drow2, isem2)
    pltpu.async_copy(h_hbm.at[srow0], rin0, gsem0)
    pltpu.async_copy(h_hbm.at[srow1], rin1, gsem1)

    def zden(r, _):
        for g in range(8):
            denv[r, pl.ds(g * 16, 16)] = jnp.zeros((16,), jnp.float32)
        return 0
    lax.fori_loop(0, DROWS, zden, 0)
    for i in range(DROWS // 16):
        iden[pl.ds(i * 16, 16)] = i * 16 + lane

    nz = (ZCH - s + NS - 1) // NS

    def zacc(i, _):
        t = s + i * NS
        pltpu.sync_copy(denv, accm.at[pl.ds(t * DROWS, DROWS)])
        return 0
    lax.fori_loop(0, nz, zacc, 0)

    @pl.when(s == 0)
    def _():
        pltpu.sync_copy(denv, accd)

    plsc.subcore_barrier()

    def compute_w(b, srw, drw, wbk):
        for g in range(SB // 16):
            sidx = srw[pl.ds(g * 16, 16)]
            didx = drw[pl.ds(g * 16, 16)]
            ps = plsc.bitcast(plsc.load_gather(apv, [sidx]), jnp.bfloat16)
            pd = plsc.bitcast(plsc.load_gather(apv, [didx]), jnp.bfloat16)
            e = (plsc.unpack(ps, format=plsc.PackFormat.INTERLEAVED)[0]
                 + plsc.unpack(pd, format=plsc.PackFormat.INTERLEAVED)[1])
            e = jnp.where(e > 0, e, e * jnp.float32(0.2))
            w = jnp.exp(e)
            pos = (start + b) * SB + g * 16 + lane
            wbk[pl.ds(g * 16, 16)] = jnp.where(pos < EO, w, jnp.float32(0.0))

    def scale_rows(rin_k, wbk):
        @plsc.parallel_loop(0, SB, unroll=8)
        def _(r):
            ridx = jnp.full((16,), r, jnp.int32)
            wspl = plsc.load_gather(wbk, [ridx])
            for g in range(8):
                rin_k[r, pl.ds(g * 16, 16)] = (
                    rin_k[r, pl.ds(g * 16, 16)] * wspl)

    def denom_rows(wbk, drw):
        @plsc.parallel_loop(0, SB, unroll=4)
        def _(r):
            ridx = jnp.full((16,), r, jnp.int32)
            wspl = plsc.load_gather(wbk, [ridx])
            dsp = plsc.load_gather(drw, [ridx])
            plsc.addupdate_scatter(
                denv,
                [lax.shift_right_logical(dsp, 7),
                 lax.bitwise_and(dsp, jnp.int32(127))],
                wspl, mask=lane0)

    bufs = ((srow0, drow0, rin0, wb0, sx0, gsem0, isem0, ssem0),
            (srow1, drow1, rin1, wb1, sx1, gsem1, isem1, ssem1),
            (srow2, drow2, rin2, wb2, sx2, gsem2, isem2, ssem2))

    def body(jj, _):
        for k in range(3):
            b = jj * 3 + k
            srw, drw, rin_k, wbk, sxk, gsem_k, isem_k, ssem_k = bufs[k]
            (srw2, drw2, rin2_, _, sx2_, gsem2_, isem2_,
             ssem2_) = bufs[(k + 2) % 3]

            compute_w(b, srw, drw, wbk)

            @pl.when(b + 2 < nb)
            def _():
                pltpu.make_async_copy(src_hbm.at[start + b + 2, 0], srw2,
                                      isem2_).wait()
                pltpu.make_async_copy(dst_hbm.at[start + b + 2, 0], drw2,
                                      isem2_).wait()

                @pl.when(b >= 1)
                def _():
                    pltpu.make_async_copy(rin2_, accm.at[sx2_],
                                          ssem2_).wait()

                pltpu.async_copy(h_hbm.at[srw2], rin2_, gsem2_)

            pltpu.make_async_copy(h_hbm.at[srw], rin_k, gsem_k).wait()
            scale_rows(rin_k, wbk)
            for g in range(SB // 16):
                sxk[pl.ds(g * 16, 16)] = drw[pl.ds(g * 16, 16)]
            pltpu.async_copy(rin_k, accm.at[sxk], ssem_k, add=True)
            denom_rows(wbk, drw)

            @pl.when(b + 3 < nb)
            def _():
                pltpu.async_copy(src_hbm.at[start + b + 3, 0], srw, isem_k)
                pltpu.async_copy(dst_hbm.at[start + b + 3, 0], drw, isem_k)
        return 0

    lax.fori_loop(0, nb // 3, body, 0)
    pltpu.make_async_copy(rin0, accm.at[sx0], ssem0).wait()
    pltpu.make_async_copy(rin1, accm.at[sx1], ssem1).wait()
    pltpu.make_async_copy(rin2, accm.at[sx2], ssem2).wait()

    pltpu.sync_copy(denv, accd.at[iden], add=True)

    plsc.subcore_barrier()

    def cout(i, _):
        t = s + i * NS
        pltpu.sync_copy(accm.at[pl.ds(t * DROWS, DROWS)],
                        outm_hbm.at[c, pl.ds(t * DROWS, DROWS)])
        return 0
    lax.fori_loop(0, nz, cout, 0)

    @pl.when(s == 0)
    def _():
        pltpu.sync_copy(accd, outd_hbm.at[c])


_agg_call = pl.kernel(
    _agg_body,
    out_type=(jax.ShapeDtypeStruct((2, N, CDIM), jnp.float32),
              jax.ShapeDtypeStruct((2, DROWS, 128), jnp.float32)),
    mesh=plsc.VectorSubcoreMesh(core_axis_name="c", subcore_axis_name="s"),
    compiler_params=pltpu.CompilerParams(needs_layout_passes=False),
    scratch_types=[
        pltpu.VMEM((SB,), jnp.int32),
        pltpu.VMEM((SB,), jnp.int32),
        pltpu.VMEM((SB,), jnp.int32),
        pltpu.VMEM((SB,), jnp.int32),
        pltpu.VMEM((SB,), jnp.int32),
        pltpu.VMEM((SB,), jnp.int32),
        pltpu.VMEM((N,), jnp.float32),
        pltpu.VMEM((SB, CDIM), jnp.float32),
        pltpu.VMEM((SB, CDIM), jnp.float32),
        pltpu.VMEM((SB, CDIM), jnp.float32),
        pltpu.VMEM((SB,), jnp.float32),
        pltpu.VMEM((SB,), jnp.float32),
        pltpu.VMEM((SB,), jnp.float32),
        pltpu.VMEM((DROWS, 128), jnp.float32),
        pltpu.VMEM((DROWS,), jnp.int32),
        pltpu.VMEM((SB,), jnp.int32),
        pltpu.VMEM((SB,), jnp.int32),
        pltpu.VMEM((SB,), jnp.int32),
        pltpu.SemaphoreType.DMA,
        pltpu.SemaphoreType.DMA,
        pltpu.SemaphoreType.DMA,
        pltpu.SemaphoreType.DMA,
        pltpu.SemaphoreType.DMA,
        pltpu.SemaphoreType.DMA,
        pltpu.SemaphoreType.DMA,
        pltpu.SemaphoreType.DMA,
        pltpu.SemaphoreType.DMA,
        pltpu.VMEM_SHARED((N, CDIM), jnp.float32),
        pltpu.VMEM_SHARED((DROWS, 128), jnp.float32),
    ],
)


@jax.jit
def kernel(x, edge_index, batch, W1, att_src1, att_dst1, b1,
           W2, att_src2, att_dst2, b2, lin_W, lin_b):
    loop = jnp.arange(N, dtype=edge_index.dtype)
    src = jnp.concatenate([edge_index[0], loop])
    dst = jnp.concatenate([edge_index[1], loop])
    srcp = jnp.pad(src, (0, EP - EO)).reshape(NSB, 1, SB)
    dstp = jnp.pad(dst, (0, EP - EO)).reshape(NSB, 1, SB)

    def pack_a(a_s, a_d):
        ab = jnp.stack([a_s.reshape(N).astype(jnp.bfloat16),
                        a_d.reshape(N).astype(jnp.bfloat16)], axis=-1)
        return lax.bitcast_convert_type(ab, jnp.float32)

    h1, a_s1, a_d1 = _embed(x, W1, att_src1.reshape(1, CDIM),
                            att_dst1.reshape(1, CDIM))
    p1, d1 = _agg_call(srcp, dstp, h1, pack_a(a_s1, a_d1))
    h2, a_s2, a_d2 = _combine_embed(p1, d1.reshape(2, DROWS * 128, 1),
                                    b1.reshape(1, CDIM), W2,
                                    att_src2.reshape(1, CDIM),
                                    att_dst2.reshape(1, CDIM))
    p2, d2 = _agg_call(srcp, dstp, h2, pack_a(a_s2, a_d2))
    return _pool(p2, d2.reshape(2, DROWS * 128, 1), b2.reshape(1, CDIM),
                 batch.reshape(N, 1), lin_W, lin_b.reshape(1, CDIM))
